# baseline (speedup 1.0000x reference)
import functools

import jax
import jax.numpy as jnp
from jax import lax
from jax.experimental import pallas as pl
from jax.experimental.pallas import tpu as pltpu
from jax.experimental.pallas import tpu_sc as plsc

_K = 32
_B = 32
_S = 8
_N = 32768
_LN = 16
_CAP = 4096
_COMPACT_AT = 2048
_CPB = 128
_NBLK = _N // (_CPB * _LN)
_NEG = float("-inf")
_BIGI = 1 << 30


def _iota():
    return lax.iota(jnp.int32, _LN)


def _lane_top2_threshold(row, cand, off):
    iota = _iota()
    neg = jnp.full((_LN,), _NEG)

    nv = (off + _LN - 1) // _LN

    @plsc.parallel_loop(0, nv * _LN, step=_LN, carry=(neg, neg))
    def m_body(base, carry):
        m1, m2 = carry
        valid = (base + iota) < off
        idxv = jnp.where(valid, cand[pl.ds(base, _LN)], 0)
        v = jnp.where(valid, plsc.load_gather(row, [idxv]), _NEG)
        a = jnp.minimum(m1, v)
        m1 = jnp.maximum(m1, v)
        m2 = jnp.maximum(m2, a)
        return m1, m2

    _, m2 = m_body
    return jnp.full((_LN,), jnp.min(m2))


def _scatter_compact(ref, dst, x):
    plsc.store_scatter(ref, [dst], x)


def _refilter(row, cand, off, thr, vstash):
    iota = _iota()
    dump = _CAP + iota

    def f_body(j, noffv):
        base = j * _LN
        valid = (base + iota) < off
        idxv = cand[pl.ds(base, _LN)]
        safe = jnp.where(valid, idxv, 0)
        v = plsc.load_gather(row, [safe])
        keep = valid & (v >= thr)
        ki = keep.astype(jnp.int32)
        cum = jnp.cumsum(ki)
        dst = jnp.where(keep, noffv + cum - ki, dump)
        _scatter_compact(cand, dst, idxv)
        if vstash is not None:
            _scatter_compact(vstash, dst, v)
        return noffv + plsc.all_reduce_population_count(keep)

    nv = (off + _LN - 1) // _LN
    noffv = lax.fori_loop(0, nv, f_body, jnp.zeros((_LN,), jnp.int32))
    return jnp.max(noffv)


def _process_row(row, cand, vstash, gbuf, vbuf, srow):
    iota = _iota()
    neg = jnp.full((_LN,), _NEG)

    @plsc.parallel_loop(0, _CPB * _LN, step=_LN, unroll=8, carry=(neg, neg))
    def t2_loop(pos, carry):
        m1, m2 = carry
        v = row[pl.ds(pos, _LN)]
        a = jnp.minimum(m1, v)
        return jnp.maximum(m1, v), jnp.maximum(m2, a)

    _, m2 = t2_loop
    thr0 = jnp.full((_LN,), jnp.min(m2))

    dump = _CAP + iota

    def blk_body(blk, carry):
        offv0, pos0, ivec0, thr = carry

        @plsc.parallel_loop(0, _CPB, step=1, unroll=16,
                            carry=(offv0, pos0, ivec0))
        def floop(c, fc):
            offv, pos, ivec = fc
            v = row[pl.ds(pos, _LN)]
            m = v >= thr
            mi = m.astype(jnp.int32)
            cum = jnp.cumsum(mi)
            offc = jnp.minimum(offv, _CAP - _LN)
            dst = jnp.where(m, offc + cum - mi, dump)
            _scatter_compact(cand, dst, ivec)
            cnt = plsc.all_reduce_population_count(m)
            return offv + cnt, pos + _LN, ivec + _LN

        offv, pos, ivec = floop

        def compact(args):
            offv, _ = args
            off = jnp.max(offv)
            nthr = _lane_top2_threshold(row, cand, off)
            noff = _refilter(row, cand, off, nthr, None)
            return jnp.full((_LN,), noff), nthr

        offv, thr = lax.cond(jnp.max(offv) >= _COMPACT_AT, compact,
                             lambda args: args, (offv, thr))
        return offv, pos, ivec, thr

    offv, _, _, _ = lax.fori_loop(
        0, _NBLK, blk_body,
        (jnp.zeros((_LN,), jnp.int32), jnp.int32(0), iota, thr0))
    off = jnp.max(offv)

    thr = _lane_top2_threshold(row, cand, off)
    off = _refilter(row, cand, off, thr, vstash)

    nv = (off + _LN - 1) // _LN
    neg = jnp.full((_LN,), _NEG)
    bigv = jnp.full((_LN,), _BIGI)

    def k_body(k, carry):
        g_lo, g_hi, v_lo, v_hi = carry

        def p_max(j, acc):
            base = j * _LN
            valid = (base + iota) < off
            v = jnp.where(valid, vstash[pl.ds(base, _LN)], _NEG)
            return jnp.maximum(acc, v)

        s = jnp.max(lax.fori_loop(0, nv, p_max, neg))
        sv = jnp.full((_LN,), s)

        def p_min(j, bi):
            base = j * _LN
            valid = (base + iota) < off
            v = vstash[pl.ds(base, _LN)]
            g = cand[pl.ds(base, _LN)]
            hit = valid & (v == sv)
            return jnp.minimum(bi, jnp.where(hit, g, _BIGI))

        gm = jnp.min(lax.fori_loop(0, nv, p_min, bigv))
        gv = jnp.full((_LN,), gm)

        def p_mask(j, _):
            base = j * _LN
            g = cand[pl.ds(base, _LN)]
            v = vstash[pl.ds(base, _LN)]
            vstash[pl.ds(base, _LN)] = jnp.where(g == gv, _NEG, v)
            return 0

        lax.fori_loop(0, nv, p_mask, 0)

        sel = iota == (k % _LN)
        lo = k < _LN
        g_lo = jnp.where(sel & lo, gm, g_lo)
        g_hi = jnp.where(sel & (~lo), gm, g_hi)
        v_lo = jnp.where(sel & lo, s, v_lo)
        v_hi = jnp.where(sel & (~lo), s, v_hi)
        return g_lo, g_hi, v_lo, v_hi

    zi = jnp.zeros((_LN,), jnp.int32)
    zf = jnp.zeros((_LN,), jnp.float32)
    g_lo, g_hi, v_lo, v_hi = lax.fori_loop(0, _K, k_body, (zi, zi, zf, zf))
    gbuf[pl.ds(srow * _K, _LN)] = g_lo
    gbuf[pl.ds(srow * _K + _LN, _LN)] = g_hi
    vbuf[pl.ds(srow * _K, _LN)] = v_lo
    vbuf[pl.ds(srow * _K + _LN, _LN)] = v_hi


def _make_sc_kernel():
    mesh = plsc.VectorSubcoreMesh(core_axis_name="c", subcore_axis_name="s")

    @functools.partial(
        pl.kernel,
        mesh=mesh,
        compiler_params=pltpu.CompilerParams(needs_layout_passes=False),
        out_type=jax.ShapeDtypeStruct((_B, _LN), jnp.int32),
        scratch_types=[
            pltpu.VMEM((_N,), jnp.float32),
            pltpu.VMEM((_N,), jnp.float32),
            pltpu.VMEM((_CAP + _LN,), jnp.int32),
            pltpu.VMEM((_CAP + _LN,), jnp.float32),
            pltpu.VMEM((_S * _K,), jnp.int32),
            pltpu.VMEM((_S * _K,), jnp.float32),
            pltpu.VMEM((_N,), jnp.float32),
            pltpu.VMEM((_LN,), jnp.int32),
            pltpu.SemaphoreType.DMA,
            pltpu.SemaphoreType.DMA,
        ],
    )
    def sc_kernel(in_hbm, out_hbm, rowa, rowb, cand, vstash, gbuf, vbuf,
                  table, outv, sema, semb):
        wid = lax.axis_index("s") * 2 + lax.axis_index("c")
        rows = (rowa, rowb)
        sems = (sema, semb)
        pltpu.async_copy(in_hbm.at[wid, 0], rowa, sema)
        for s in range(_S):
            if s + 1 < _S:
                pltpu.async_copy(in_hbm.at[wid, s + 1],
                                 rows[(s + 1) % 2], sems[(s + 1) % 2])
            pltpu.make_async_copy(in_hbm.at[wid, s],
                                  rows[s % 2], sems[s % 2]).wait()
            _process_row(rows[s % 2], cand, vstash, gbuf, vbuf, s)

        nseg = (_S * _K) // _LN

        def p_zero(j, _):
            g = gbuf[pl.ds(j * _LN, _LN)]
            plsc.store_scatter(table, [g], jnp.zeros((_LN,), jnp.float32))
            return 0

        lax.fori_loop(0, nseg, p_zero, 0)

        def p_add(j, _):
            g = gbuf[pl.ds(j * _LN, _LN)]
            v = vbuf[pl.ds(j * _LN, _LN)]
            plsc.addupdate_scatter(table, [g], v)
            return 0

        lax.fori_loop(0, nseg, p_add, 0)

        def p_best(j, acc):
            g = gbuf[pl.ds(j * _LN, _LN)]
            sc = plsc.load_gather(table, [g])
            return jnp.maximum(acc, sc)

        best = jnp.max(lax.fori_loop(0, nseg, p_best,
                                     jnp.full((_LN,), _NEG)))
        bestv = jnp.full((_LN,), best)

        def p_pick(j, bi):
            g = gbuf[pl.ds(j * _LN, _LN)]
            sc = plsc.load_gather(table, [g])
            return jnp.minimum(bi, jnp.where(sc == bestv, g, _BIGI))

        gwin = jnp.min(lax.fori_loop(0, nseg, p_pick,
                                     jnp.full((_LN,), _BIGI)))
        outv[...] = jnp.full((_LN,), gwin)
        pltpu.sync_copy(outv, out_hbm.at[wid])

    return sc_kernel


_SC_KERNEL = _make_sc_kernel()


def kernel(inputs):
    out = _SC_KERNEL(inputs)
    return out[:, 0]


# speedup vs baseline: 1.3241x; 1.3241x over previous
import functools

import jax
import jax.numpy as jnp
from jax import lax
from jax.experimental import pallas as pl
from jax.experimental.pallas import tpu as pltpu
from jax.experimental.pallas import tpu_sc as plsc

_K = 32
_B = 32
_S = 8
_N = 32768
_LN = 16
_CAP = 4096
_COMPACT_AT = 2048
_CPB = 128
_NBLK = _N // (_CPB * _LN)
_T2B = 2
_NVREG = 6
_NEG = float("-inf")
_BIGI = 1 << 30


def _iota():
    return lax.iota(jnp.int32, _LN)


def _safe_idx(idxv):
    return jnp.minimum(jnp.maximum(idxv, 0), _N - 1)


def _lane_top2_threshold(row, cand, off):
    iota = _iota()
    neg = jnp.full((_LN,), _NEG)
    nv = (off + _LN - 1) // _LN

    @plsc.parallel_loop(0, nv * _LN, step=_LN, carry=(neg, neg))
    def m_body(base, carry):
        m1, m2 = carry
        valid = (base + iota) < off
        idxv = _safe_idx(cand[pl.ds(base, _LN)])
        v = jnp.where(valid, plsc.load_gather(row, [idxv]), _NEG)
        a = jnp.minimum(m1, v)
        m1 = jnp.maximum(m1, v)
        m2 = jnp.maximum(m2, a)
        return m1, m2

    _, m2 = m_body
    return jnp.full((_LN,), jnp.min(m2))


def _refilter(row, cand, off, thr, vstash):
    iota = _iota()

    def f_body(j, noffv):
        base = j * _LN
        valid = (base + iota) < off
        idxv = cand[pl.ds(base, _LN)]
        v = plsc.load_gather(row, [_safe_idx(idxv)])
        keep = valid & (v >= thr)
        ki = keep.astype(jnp.int32)
        dst = noffv + jnp.cumsum(ki) - ki
        plsc.store_scatter(cand, [dst], idxv, mask=keep)
        if vstash is not None:
            plsc.store_scatter(vstash, [dst], v, mask=keep)
        return noffv + plsc.all_reduce_population_count(keep)

    nv = (off + _LN - 1) // _LN
    noffv = lax.fori_loop(0, nv, f_body, jnp.zeros((_LN,), jnp.int32))
    return jnp.max(noffv)


def _process_row(row, cand, vstash, gbuf, vbuf, srow):
    iota = _iota()
    neg = jnp.full((_LN,), _NEG)
    bigv = jnp.full((_LN,), _BIGI)

    @plsc.parallel_loop(0, _T2B * _CPB * _LN, step=_LN, unroll=8,
                        carry=(neg, neg))
    def t2_loop(pos, carry):
        m1, m2 = carry
        v = row[pl.ds(pos, _LN)]
        a = jnp.minimum(m1, v)
        return jnp.maximum(m1, v), jnp.maximum(m2, a)

    _, m2 = t2_loop
    thr0 = jnp.full((_LN,), jnp.min(m2))

    def blk_body(blk, carry):
        offv0, pos0, ivec0, thr = carry

        @plsc.parallel_loop(0, _CPB, step=1, unroll=8,
                            carry=(offv0, pos0, ivec0))
        def floop(c, fc):
            offv, pos, ivec = fc
            v = row[pl.ds(pos, _LN)]
            m = v >= thr
            mi = m.astype(jnp.int32)
            dst = offv + jnp.cumsum(mi) - mi
            plsc.store_scatter(cand, [dst], ivec, mask=m)
            cnt = plsc.all_reduce_population_count(m)
            offv = jnp.minimum(offv + cnt, _CAP - _LN)
            return offv, pos + _LN, ivec + _LN

        offv, pos, ivec = floop

        def compact(args):
            offv, _ = args
            off = jnp.max(offv)
            nthr = _lane_top2_threshold(row, cand, off)
            noff = _refilter(row, cand, off, nthr, None)
            return jnp.full((_LN,), noff), nthr

        offv, thr = lax.cond(jnp.max(offv) >= _COMPACT_AT, compact,
                             lambda args: args, (offv, thr))
        return offv, pos, ivec, thr

    offv, _, _, _ = lax.fori_loop(
        0, _NBLK, blk_body,
        (jnp.zeros((_LN,), jnp.int32), jnp.int32(0), iota, thr0))
    off = jnp.max(offv)

    thr = _lane_top2_threshold(row, cand, off)
    off = _refilter(row, cand, off, thr, vstash)

    zi = jnp.zeros((_LN,), jnp.int32)
    zf = jnp.zeros((_LN,), jnp.float32)

    def extract_fast(off):
        Vs, Gs = [], []
        for j in range(_NVREG):
            base = j * _LN
            valid = (base + iota) < off
            Vs.append(jnp.where(valid, vstash[pl.ds(base, _LN)], _NEG))
            Gs.append(jnp.where(valid, cand[pl.ds(base, _LN)], _BIGI))
        Gs = tuple(Gs)

        def k_body(k, carry):
            g_lo, g_hi, v_lo, v_hi, vt = carry
            t = vt[0]
            for j in range(1, _NVREG):
                t = jnp.maximum(t, vt[j])
            s = jnp.max(t)
            sv = jnp.full((_LN,), s)
            w = bigv
            for j in range(_NVREG):
                w = jnp.minimum(w, jnp.where(vt[j] == sv, Gs[j], _BIGI))
            gm = jnp.min(w)
            gv = jnp.full((_LN,), gm)
            vt = tuple(jnp.where(Gs[j] == gv, _NEG, vt[j])
                       for j in range(_NVREG))
            sel = iota == (k % _LN)
            lo = k < _LN
            g_lo = jnp.where(sel & lo, gm, g_lo)
            g_hi = jnp.where(sel & (~lo), gm, g_hi)
            v_lo = jnp.where(sel & lo, s, v_lo)
            v_hi = jnp.where(sel & (~lo), s, v_hi)
            return g_lo, g_hi, v_lo, v_hi, vt

        g_lo, g_hi, v_lo, v_hi, _ = lax.fori_loop(
            0, _K, k_body, (zi, zi, zf, zf, tuple(Vs)))
        return g_lo, g_hi, v_lo, v_hi

    def extract_slow(off):
        nv = (off + _LN - 1) // _LN

        def k_body(k, carry):
            g_lo, g_hi, v_lo, v_hi = carry

            def p_max(j, acc):
                base = j * _LN
                valid = (base + iota) < off
                v = jnp.where(valid, vstash[pl.ds(base, _LN)], _NEG)
                return jnp.maximum(acc, v)

            s = jnp.max(lax.fori_loop(0, nv, p_max, neg))
            sv = jnp.full((_LN,), s)

            def p_min(j, bi):
                base = j * _LN
                valid = (base + iota) < off
                v = vstash[pl.ds(base, _LN)]
                g = cand[pl.ds(base, _LN)]
                hit = valid & (v == sv)
                return jnp.minimum(bi, jnp.where(hit, g, _BIGI))

            gm = jnp.min(lax.fori_loop(0, nv, p_min, bigv))
            gv = jnp.full((_LN,), gm)

            def p_mask(j, _):
                base = j * _LN
                g = cand[pl.ds(base, _LN)]
                v = vstash[pl.ds(base, _LN)]
                vstash[pl.ds(base, _LN)] = jnp.where(g == gv, _NEG, v)
                return 0

            lax.fori_loop(0, nv, p_mask, 0)

            sel = iota == (k % _LN)
            lo = k < _LN
            g_lo = jnp.where(sel & lo, gm, g_lo)
            g_hi = jnp.where(sel & (~lo), gm, g_hi)
            v_lo = jnp.where(sel & lo, s, v_lo)
            v_hi = jnp.where(sel & (~lo), s, v_hi)
            return g_lo, g_hi, v_lo, v_hi

        return lax.fori_loop(0, _K, k_body, (zi, zi, zf, zf))

    g_lo, g_hi, v_lo, v_hi = lax.cond(off <= _NVREG * _LN,
                                      extract_fast, extract_slow, off)
    gbuf[pl.ds(srow * _K, _LN)] = g_lo
    gbuf[pl.ds(srow * _K + _LN, _LN)] = g_hi
    vbuf[pl.ds(srow * _K, _LN)] = v_lo
    vbuf[pl.ds(srow * _K + _LN, _LN)] = v_hi


def _make_sc_kernel():
    mesh = plsc.VectorSubcoreMesh(core_axis_name="c", subcore_axis_name="s")

    @functools.partial(
        pl.kernel,
        mesh=mesh,
        compiler_params=pltpu.CompilerParams(needs_layout_passes=False),
        out_type=jax.ShapeDtypeStruct((_B, _LN), jnp.int32),
        scratch_types=[
            pltpu.VMEM((_N,), jnp.float32),
            pltpu.VMEM((_N,), jnp.float32),
            pltpu.VMEM((_CAP + _LN,), jnp.int32),
            pltpu.VMEM((_CAP + _LN,), jnp.float32),
            pltpu.VMEM((_S * _K,), jnp.int32),
            pltpu.VMEM((_S * _K,), jnp.float32),
            pltpu.VMEM((_N,), jnp.float32),
            pltpu.VMEM((_LN,), jnp.int32),
            pltpu.SemaphoreType.DMA,
            pltpu.SemaphoreType.DMA,
        ],
    )
    def sc_kernel(in_hbm, out_hbm, rowa, rowb, cand, vstash, gbuf, vbuf,
                  table, outv, sema, semb):
        wid = lax.axis_index("s") * 2 + lax.axis_index("c")
        rows = (rowa, rowb)
        sems = (sema, semb)
        pltpu.async_copy(in_hbm.at[wid, 0], rowa, sema)
        for s in range(_S):
            if s + 1 < _S:
                pltpu.async_copy(in_hbm.at[wid, s + 1],
                                 rows[(s + 1) % 2], sems[(s + 1) % 2])
            pltpu.make_async_copy(in_hbm.at[wid, s],
                                  rows[s % 2], sems[s % 2]).wait()
            _process_row(rows[s % 2], cand, vstash, gbuf, vbuf, s)

        nseg = (_S * _K) // _LN

        def _gseg(j):
            g = gbuf[pl.ds(j * _LN, _LN)]
            return jnp.minimum(jnp.maximum(g, 0), _N - 1)

        def p_zero(j, _):
            plsc.store_scatter(table, [_gseg(j)],
                               jnp.zeros((_LN,), jnp.float32))
            return 0

        lax.fori_loop(0, nseg, p_zero, 0)

        def p_add(j, _):
            v = vbuf[pl.ds(j * _LN, _LN)]
            plsc.addupdate_scatter(table, [_gseg(j)], v)
            return 0

        lax.fori_loop(0, nseg, p_add, 0)

        def p_best(j, acc):
            sc = plsc.load_gather(table, [_gseg(j)])
            return jnp.maximum(acc, sc)

        best = jnp.max(lax.fori_loop(0, nseg, p_best,
                                     jnp.full((_LN,), _NEG)))
        bestv = jnp.full((_LN,), best)

        def p_pick(j, bi):
            g = _gseg(j)
            sc = plsc.load_gather(table, [g])
            return jnp.minimum(bi, jnp.where(sc == bestv, g, _BIGI))

        gwin = jnp.min(lax.fori_loop(0, nseg, p_pick,
                                     jnp.full((_LN,), _BIGI)))
        outv[...] = jnp.full((_LN,), gwin)
        pltpu.sync_copy(outv, out_hbm.at[wid])

    return sc_kernel


_SC_KERNEL = _make_sc_kernel()


def kernel(inputs):
    out = _SC_KERNEL(inputs)
    return out[:, 0]


# speedup vs baseline: 1.3333x; 1.0069x over previous
import functools

import jax
import jax.numpy as jnp
from jax import lax
from jax.experimental import pallas as pl
from jax.experimental.pallas import tpu as pltpu
from jax.experimental.pallas import tpu_sc as plsc

_K = 32
_B = 32
_S = 8
_N = 32768
_LN = 16
_CAP = 4096
_COMPACT_AT = 2048
_CPB = 128
_NBLK = _N // (_CPB * _LN)
_T2B = 2
_NVREG = 6
_NEG = float("-inf")
_BIGI = 1 << 30


def _iota():
    return lax.iota(jnp.int32, _LN)


def _safe_idx(idxv):
    return jnp.minimum(jnp.maximum(idxv, 0), _N - 1)


def _lane_top2_threshold(row, cand, off):
    iota = _iota()
    neg = jnp.full((_LN,), _NEG)
    nv = (off + _LN - 1) // _LN

    @plsc.parallel_loop(0, nv * _LN, step=_LN, carry=(neg, neg))
    def m_body(base, carry):
        m1, m2 = carry
        valid = (base + iota) < off
        idxv = _safe_idx(cand[pl.ds(base, _LN)])
        v = jnp.where(valid, plsc.load_gather(row, [idxv]), _NEG)
        a = jnp.minimum(m1, v)
        m1 = jnp.maximum(m1, v)
        m2 = jnp.maximum(m2, a)
        return m1, m2

    _, m2 = m_body
    return jnp.full((_LN,), jnp.min(m2))


def _refilter(row, cand, off, thr, vstash):
    iota = _iota()

    def f_body(j, noffv):
        base = j * _LN
        valid = (base + iota) < off
        idxv = cand[pl.ds(base, _LN)]
        v = plsc.load_gather(row, [_safe_idx(idxv)])
        keep = valid & (v >= thr)
        ki = keep.astype(jnp.int32)
        dst = noffv + jnp.cumsum(ki) - ki
        plsc.store_scatter(cand, [dst], idxv, mask=keep)
        if vstash is not None:
            plsc.store_scatter(vstash, [dst], v, mask=keep)
        return noffv + plsc.all_reduce_population_count(keep)

    nv = (off + _LN - 1) // _LN
    noffv = lax.fori_loop(0, nv, f_body, jnp.zeros((_LN,), jnp.int32))
    return jnp.max(noffv)


def _process_row(row, cand, vstash, gbuf, vbuf, srow):
    iota = _iota()
    neg = jnp.full((_LN,), _NEG)
    bigv = jnp.full((_LN,), _BIGI)

    @plsc.parallel_loop(0, _T2B * _CPB * _LN, step=_LN, unroll=8,
                        carry=(neg, neg))
    def t2_loop(pos, carry):
        m1, m2 = carry
        v = row[pl.ds(pos, _LN)]
        a = jnp.minimum(m1, v)
        return jnp.maximum(m1, v), jnp.maximum(m2, a)

    _, m2 = t2_loop
    thr0 = jnp.full((_LN,), jnp.min(m2))

    def blk_body(blk, carry):
        offv0, pos0, ivec0, thr = carry

        @plsc.parallel_loop(0, _CPB, step=1, unroll=8,
                            carry=(offv0, pos0, ivec0))
        def floop(c, fc):
            offv, pos, ivec = fc
            v = row[pl.ds(pos, _LN)]
            m = v >= thr
            mi = m.astype(jnp.int32)
            dst = jnp.minimum(offv, _CAP - _LN) + jnp.cumsum(mi) - mi
            plsc.store_scatter(cand, [dst], ivec, mask=m)
            cnt = plsc.all_reduce_population_count(m)
            return offv + cnt, pos + _LN, ivec + _LN

        offv, pos, ivec = floop

        def compact(args):
            offv, _ = args
            off = jnp.max(offv)
            nthr = _lane_top2_threshold(row, cand, off)
            noff = _refilter(row, cand, off, nthr, None)
            return jnp.full((_LN,), noff), nthr

        offv, thr = lax.cond(jnp.max(offv) >= _COMPACT_AT, compact,
                             lambda args: args, (offv, thr))
        return offv, pos, ivec, thr

    offv, _, _, _ = lax.fori_loop(
        0, _NBLK, blk_body,
        (jnp.zeros((_LN,), jnp.int32), jnp.int32(0), iota, thr0))
    off = jnp.max(offv)

    thr = _lane_top2_threshold(row, cand, off)
    off = _refilter(row, cand, off, thr, vstash)

    zi = jnp.zeros((_LN,), jnp.int32)
    zf = jnp.zeros((_LN,), jnp.float32)

    def extract_fast(off):
        Vs, Gs = [], []
        for j in range(_NVREG):
            base = j * _LN
            valid = (base + iota) < off
            Vs.append(jnp.where(valid, vstash[pl.ds(base, _LN)], _NEG))
            Gs.append(jnp.where(valid, cand[pl.ds(base, _LN)], _BIGI))
        Gs = tuple(Gs)

        def k_body(k, carry):
            g_lo, g_hi, v_lo, v_hi, vt = carry
            t = vt[0]
            for j in range(1, _NVREG):
                t = jnp.maximum(t, vt[j])
            s = jnp.max(t)
            sv = jnp.full((_LN,), s)
            w = bigv
            for j in range(_NVREG):
                w = jnp.minimum(w, jnp.where(vt[j] == sv, Gs[j], _BIGI))
            gm = jnp.min(w)
            gv = jnp.full((_LN,), gm)
            vt = tuple(jnp.where(Gs[j] == gv, _NEG, vt[j])
                       for j in range(_NVREG))
            sel = iota == (k % _LN)
            lo = k < _LN
            g_lo = jnp.where(sel & lo, gm, g_lo)
            g_hi = jnp.where(sel & (~lo), gm, g_hi)
            v_lo = jnp.where(sel & lo, s, v_lo)
            v_hi = jnp.where(sel & (~lo), s, v_hi)
            return g_lo, g_hi, v_lo, v_hi, vt

        g_lo, g_hi, v_lo, v_hi, _ = lax.fori_loop(
            0, _K, k_body, (zi, zi, zf, zf, tuple(Vs)))
        return g_lo, g_hi, v_lo, v_hi

    def extract_slow(off):
        nv = (off + _LN - 1) // _LN

        def k_body(k, carry):
            g_lo, g_hi, v_lo, v_hi = carry

            def p_max(j, acc):
                base = j * _LN
                valid = (base + iota) < off
                v = jnp.where(valid, vstash[pl.ds(base, _LN)], _NEG)
                return jnp.maximum(acc, v)

            s = jnp.max(lax.fori_loop(0, nv, p_max, neg))
            sv = jnp.full((_LN,), s)

            def p_min(j, bi):
                base = j * _LN
                valid = (base + iota) < off
                v = vstash[pl.ds(base, _LN)]
                g = cand[pl.ds(base, _LN)]
                hit = valid & (v == sv)
                return jnp.minimum(bi, jnp.where(hit, g, _BIGI))

            gm = jnp.min(lax.fori_loop(0, nv, p_min, bigv))
            gv = jnp.full((_LN,), gm)

            def p_mask(j, _):
                base = j * _LN
                g = cand[pl.ds(base, _LN)]
                v = vstash[pl.ds(base, _LN)]
                vstash[pl.ds(base, _LN)] = jnp.where(g == gv, _NEG, v)
                return 0

            lax.fori_loop(0, nv, p_mask, 0)

            sel = iota == (k % _LN)
            lo = k < _LN
            g_lo = jnp.where(sel & lo, gm, g_lo)
            g_hi = jnp.where(sel & (~lo), gm, g_hi)
            v_lo = jnp.where(sel & lo, s, v_lo)
            v_hi = jnp.where(sel & (~lo), s, v_hi)
            return g_lo, g_hi, v_lo, v_hi

        return lax.fori_loop(0, _K, k_body, (zi, zi, zf, zf))

    g_lo, g_hi, v_lo, v_hi = lax.cond(off <= _NVREG * _LN,
                                      extract_fast, extract_slow, off)
    gbuf[pl.ds(srow * _K, _LN)] = g_lo
    gbuf[pl.ds(srow * _K + _LN, _LN)] = g_hi
    vbuf[pl.ds(srow * _K, _LN)] = v_lo
    vbuf[pl.ds(srow * _K + _LN, _LN)] = v_hi


def _make_sc_kernel():
    mesh = plsc.VectorSubcoreMesh(core_axis_name="c", subcore_axis_name="s")

    @functools.partial(
        pl.kernel,
        mesh=mesh,
        compiler_params=pltpu.CompilerParams(needs_layout_passes=False),
        out_type=jax.ShapeDtypeStruct((_B, _LN), jnp.int32),
        scratch_types=[
            pltpu.VMEM((_N,), jnp.float32),
            pltpu.VMEM((_N,), jnp.float32),
            pltpu.VMEM((_CAP + _LN,), jnp.int32),
            pltpu.VMEM((_CAP + _LN,), jnp.float32),
            pltpu.VMEM((_S * _K,), jnp.int32),
            pltpu.VMEM((_S * _K,), jnp.float32),
            pltpu.VMEM((_N,), jnp.float32),
            pltpu.VMEM((_LN,), jnp.int32),
            pltpu.SemaphoreType.DMA,
            pltpu.SemaphoreType.DMA,
        ],
    )
    def sc_kernel(in_hbm, out_hbm, rowa, rowb, cand, vstash, gbuf, vbuf,
                  table, outv, sema, semb):
        wid = lax.axis_index("s") * 2 + lax.axis_index("c")
        rows = (rowa, rowb)
        sems = (sema, semb)
        pltpu.async_copy(in_hbm.at[wid, 0], rowa, sema)
        for s in range(_S):
            if s + 1 < _S:
                pltpu.async_copy(in_hbm.at[wid, s + 1],
                                 rows[(s + 1) % 2], sems[(s + 1) % 2])
            pltpu.make_async_copy(in_hbm.at[wid, s],
                                  rows[s % 2], sems[s % 2]).wait()
            _process_row(rows[s % 2], cand, vstash, gbuf, vbuf, s)

        nseg = (_S * _K) // _LN

        def _gseg(j):
            g = gbuf[pl.ds(j * _LN, _LN)]
            return jnp.minimum(jnp.maximum(g, 0), _N - 1)

        def p_zero(j, _):
            plsc.store_scatter(table, [_gseg(j)],
                               jnp.zeros((_LN,), jnp.float32))
            return 0

        lax.fori_loop(0, nseg, p_zero, 0)

        def p_add(j, _):
            v = vbuf[pl.ds(j * _LN, _LN)]
            plsc.addupdate_scatter(table, [_gseg(j)], v)
            return 0

        lax.fori_loop(0, nseg, p_add, 0)

        def p_best(j, acc):
            sc = plsc.load_gather(table, [_gseg(j)])
            return jnp.maximum(acc, sc)

        best = jnp.max(lax.fori_loop(0, nseg, p_best,
                                     jnp.full((_LN,), _NEG)))
        bestv = jnp.full((_LN,), best)

        def p_pick(j, bi):
            g = _gseg(j)
            sc = plsc.load_gather(table, [g])
            return jnp.minimum(bi, jnp.where(sc == bestv, g, _BIGI))

        gwin = jnp.min(lax.fori_loop(0, nseg, p_pick,
                                     jnp.full((_LN,), _BIGI)))
        outv[...] = jnp.full((_LN,), gwin)
        pltpu.sync_copy(outv, out_hbm.at[wid])

    return sc_kernel


_SC_KERNEL = _make_sc_kernel()


def kernel(inputs):
    out = _SC_KERNEL(inputs)
    return out[:, 0]


# speedup vs baseline: 1.3521x; 1.0141x over previous
import functools

import jax
import jax.numpy as jnp
from jax import lax
from jax.experimental import pallas as pl
from jax.experimental.pallas import tpu as pltpu
from jax.experimental.pallas import tpu_sc as plsc

_K = 32
_B = 32
_S = 8
_N = 32768
_LN = 16
_CAP = 4096
_COMPACT_AT = 2048
_CPB = 128
_NBLK = _N // (_CPB * _LN)
_T2B = 2
_NVREG = 6
_NEG = float("-inf")
_BIGI = 1 << 30


def _iota():
    return lax.iota(jnp.int32, _LN)


def _safe_idx(idxv):
    return jnp.minimum(jnp.maximum(idxv, 0), _N - 1)


def _lane_top2_threshold(row, cand, off):
    iota = _iota()
    neg = jnp.full((_LN,), _NEG)
    nv = (off + _LN - 1) // _LN

    @plsc.parallel_loop(0, nv * _LN, step=_LN, carry=(neg, neg))
    def m_body(base, carry):
        m1, m2 = carry
        valid = (base + iota) < off
        idxv = _safe_idx(cand[pl.ds(base, _LN)])
        v = jnp.where(valid, plsc.load_gather(row, [idxv]), _NEG)
        a = jnp.minimum(m1, v)
        m1 = jnp.maximum(m1, v)
        m2 = jnp.maximum(m2, a)
        return m1, m2

    _, m2 = m_body
    return jnp.full((_LN,), jnp.min(m2))


def _refilter(row, cand, off, thr, vstash):
    iota = _iota()

    def f_body(j, noffv):
        base = j * _LN
        valid = (base + iota) < off
        idxv = cand[pl.ds(base, _LN)]
        v = plsc.load_gather(row, [_safe_idx(idxv)])
        keep = valid & (v >= thr)
        ki = keep.astype(jnp.int32)
        dst = noffv + jnp.cumsum(ki) - ki
        plsc.store_scatter(cand, [dst], idxv, mask=keep)
        if vstash is not None:
            plsc.store_scatter(vstash, [dst], v, mask=keep)
        return noffv + plsc.all_reduce_population_count(keep)

    nv = (off + _LN - 1) // _LN
    noffv = lax.fori_loop(0, nv, f_body, jnp.zeros((_LN,), jnp.int32))
    return jnp.max(noffv)


def _process_row(row, cand, vstash, gbuf, vbuf, srow):
    iota = _iota()
    neg = jnp.full((_LN,), _NEG)
    bigv = jnp.full((_LN,), _BIGI)

    @plsc.parallel_loop(0, _T2B * _CPB * _LN, step=_LN, unroll=8,
                        carry=(neg, neg))
    def t2_loop(pos, carry):
        m1, m2 = carry
        v = row[pl.ds(pos, _LN)]
        a = jnp.minimum(m1, v)
        return jnp.maximum(m1, v), jnp.maximum(m2, a)

    _, m2 = t2_loop
    thr0 = jnp.full((_LN,), jnp.min(m2))

    def blk_body(blk, carry):
        offv0, pos0, ivec0, thr = carry

        @plsc.parallel_loop(0, _CPB // 2, step=1, unroll=4,
                            carry=(offv0, pos0, ivec0))
        def floop(c, fc):
            offv, pos, ivec = fc
            v1 = row[pl.ds(pos, _LN)]
            v2 = row[pl.ds(pos + _LN, _LN)]
            m1 = v1 >= thr
            m2 = v2 >= thr
            mi1 = m1.astype(jnp.int32)
            mi2 = m2.astype(jnp.int32)
            c1 = plsc.all_reduce_population_count(m1)
            base = jnp.minimum(offv, _CAP - 2 * _LN)
            dst1 = base + jnp.cumsum(mi1) - mi1
            dst2 = base + c1 + jnp.cumsum(mi2) - mi2
            plsc.store_scatter(cand, [dst1], ivec, mask=m1)
            plsc.store_scatter(cand, [dst2], ivec + _LN, mask=m2)
            c2 = plsc.all_reduce_population_count(m2)
            return offv + (c1 + c2), pos + 2 * _LN, ivec + 2 * _LN

        offv, pos, ivec = floop

        def compact(args):
            offv, _ = args
            off = jnp.max(offv)
            nthr = _lane_top2_threshold(row, cand, off)
            noff = _refilter(row, cand, off, nthr, None)
            return jnp.full((_LN,), noff), nthr

        offv, thr = lax.cond(jnp.max(offv) >= _COMPACT_AT, compact,
                             lambda args: args, (offv, thr))
        return offv, pos, ivec, thr

    offv, _, _, _ = lax.fori_loop(
        0, _NBLK, blk_body,
        (jnp.zeros((_LN,), jnp.int32), jnp.int32(0), iota, thr0))
    off = jnp.max(offv)

    thr = _lane_top2_threshold(row, cand, off)
    off = _refilter(row, cand, off, thr, vstash)

    zi = jnp.zeros((_LN,), jnp.int32)
    zf = jnp.zeros((_LN,), jnp.float32)

    def extract_fast(off):
        Vs, Gs = [], []
        for j in range(_NVREG):
            base = j * _LN
            valid = (base + iota) < off
            Vs.append(jnp.where(valid, vstash[pl.ds(base, _LN)], _NEG))
            Gs.append(jnp.where(valid, cand[pl.ds(base, _LN)], _BIGI))
        Gs = tuple(Gs)

        def k_body(k, carry):
            g_lo, g_hi, v_lo, v_hi, vt = carry
            t = vt[0]
            for j in range(1, _NVREG):
                t = jnp.maximum(t, vt[j])
            s = jnp.max(t)
            sv = jnp.full((_LN,), s)
            w = bigv
            for j in range(_NVREG):
                w = jnp.minimum(w, jnp.where(vt[j] == sv, Gs[j], _BIGI))
            gm = jnp.min(w)
            gv = jnp.full((_LN,), gm)
            vt = tuple(jnp.where(Gs[j] == gv, _NEG, vt[j])
                       for j in range(_NVREG))
            sel = iota == (k % _LN)
            lo = k < _LN
            g_lo = jnp.where(sel & lo, gm, g_lo)
            g_hi = jnp.where(sel & (~lo), gm, g_hi)
            v_lo = jnp.where(sel & lo, s, v_lo)
            v_hi = jnp.where(sel & (~lo), s, v_hi)
            return g_lo, g_hi, v_lo, v_hi, vt

        g_lo, g_hi, v_lo, v_hi, _ = lax.fori_loop(
            0, _K, k_body, (zi, zi, zf, zf, tuple(Vs)))
        return g_lo, g_hi, v_lo, v_hi

    def extract_slow(off):
        nv = (off + _LN - 1) // _LN

        def k_body(k, carry):
            g_lo, g_hi, v_lo, v_hi = carry

            def p_max(j, acc):
                base = j * _LN
                valid = (base + iota) < off
                v = jnp.where(valid, vstash[pl.ds(base, _LN)], _NEG)
                return jnp.maximum(acc, v)

            s = jnp.max(lax.fori_loop(0, nv, p_max, neg))
            sv = jnp.full((_LN,), s)

            def p_min(j, bi):
                base = j * _LN
                valid = (base + iota) < off
                v = vstash[pl.ds(base, _LN)]
                g = cand[pl.ds(base, _LN)]
                hit = valid & (v == sv)
                return jnp.minimum(bi, jnp.where(hit, g, _BIGI))

            gm = jnp.min(lax.fori_loop(0, nv, p_min, bigv))
            gv = jnp.full((_LN,), gm)

            def p_mask(j, _):
                base = j * _LN
                g = cand[pl.ds(base, _LN)]
                v = vstash[pl.ds(base, _LN)]
                vstash[pl.ds(base, _LN)] = jnp.where(g == gv, _NEG, v)
                return 0

            lax.fori_loop(0, nv, p_mask, 0)

            sel = iota == (k % _LN)
            lo = k < _LN
            g_lo = jnp.where(sel & lo, gm, g_lo)
            g_hi = jnp.where(sel & (~lo), gm, g_hi)
            v_lo = jnp.where(sel & lo, s, v_lo)
            v_hi = jnp.where(sel & (~lo), s, v_hi)
            return g_lo, g_hi, v_lo, v_hi

        return lax.fori_loop(0, _K, k_body, (zi, zi, zf, zf))

    g_lo, g_hi, v_lo, v_hi = lax.cond(off <= _NVREG * _LN,
                                      extract_fast, extract_slow, off)
    gbuf[pl.ds(srow * _K, _LN)] = g_lo
    gbuf[pl.ds(srow * _K + _LN, _LN)] = g_hi
    vbuf[pl.ds(srow * _K, _LN)] = v_lo
    vbuf[pl.ds(srow * _K + _LN, _LN)] = v_hi


def _make_sc_kernel():
    mesh = plsc.VectorSubcoreMesh(core_axis_name="c", subcore_axis_name="s")

    @functools.partial(
        pl.kernel,
        mesh=mesh,
        compiler_params=pltpu.CompilerParams(needs_layout_passes=False),
        out_type=jax.ShapeDtypeStruct((_B, _LN), jnp.int32),
        scratch_types=[
            pltpu.VMEM((_N,), jnp.float32),
            pltpu.VMEM((_N,), jnp.float32),
            pltpu.VMEM((_CAP + _LN,), jnp.int32),
            pltpu.VMEM((_CAP + _LN,), jnp.float32),
            pltpu.VMEM((_S * _K,), jnp.int32),
            pltpu.VMEM((_S * _K,), jnp.float32),
            pltpu.VMEM((_N,), jnp.float32),
            pltpu.VMEM((_LN,), jnp.int32),
            pltpu.SemaphoreType.DMA,
            pltpu.SemaphoreType.DMA,
        ],
    )
    def sc_kernel(in_hbm, out_hbm, rowa, rowb, cand, vstash, gbuf, vbuf,
                  table, outv, sema, semb):
        wid = lax.axis_index("s") * 2 + lax.axis_index("c")
        rows = (rowa, rowb)
        sems = (sema, semb)
        pltpu.async_copy(in_hbm.at[wid, 0], rowa, sema)
        for s in range(_S):
            if s + 1 < _S:
                pltpu.async_copy(in_hbm.at[wid, s + 1],
                                 rows[(s + 1) % 2], sems[(s + 1) % 2])
            pltpu.make_async_copy(in_hbm.at[wid, s],
                                  rows[s % 2], sems[s % 2]).wait()
            _process_row(rows[s % 2], cand, vstash, gbuf, vbuf, s)

        nseg = (_S * _K) // _LN

        def _gseg(j):
            g = gbuf[pl.ds(j * _LN, _LN)]
            return jnp.minimum(jnp.maximum(g, 0), _N - 1)

        def p_zero(j, _):
            plsc.store_scatter(table, [_gseg(j)],
                               jnp.zeros((_LN,), jnp.float32))
            return 0

        lax.fori_loop(0, nseg, p_zero, 0)

        def p_add(j, _):
            v = vbuf[pl.ds(j * _LN, _LN)]
            plsc.addupdate_scatter(table, [_gseg(j)], v)
            return 0

        lax.fori_loop(0, nseg, p_add, 0)

        def p_best(j, acc):
            sc = plsc.load_gather(table, [_gseg(j)])
            return jnp.maximum(acc, sc)

        best = jnp.max(lax.fori_loop(0, nseg, p_best,
                                     jnp.full((_LN,), _NEG)))
        bestv = jnp.full((_LN,), best)

        def p_pick(j, bi):
            g = _gseg(j)
            sc = plsc.load_gather(table, [g])
            return jnp.minimum(bi, jnp.where(sc == bestv, g, _BIGI))

        gwin = jnp.min(lax.fori_loop(0, nseg, p_pick,
                                     jnp.full((_LN,), _BIGI)))
        outv[...] = jnp.full((_LN,), gwin)
        pltpu.sync_copy(outv, out_hbm.at[wid])

    return sc_kernel


_SC_KERNEL = _make_sc_kernel()


def kernel(inputs):
    out = _SC_KERNEL(inputs)
    return out[:, 0]
